# baseline (speedup 1.0000x reference)
import functools

import jax
import jax.numpy as jnp
from jax import lax
from jax.experimental import pallas as pl
from jax.experimental.pallas import tpu as pltpu
from jax.experimental.pallas import tpu_sc as plsc

B = 16384
D = 64
T = 384
NU = 1000000
NI = 100000
NC = 2
NS = 16
NW = NC * NS
RPW = B // NW
NG = RPW // 16
CH = 128
NCH = RPW // CH

BLK = 2048


@functools.cache
def _sc_gather():
    mesh = plsc.VectorSubcoreMesh(core_axis_name="c", subcore_axis_name="s")

    @functools.partial(
        pl.kernel,
        mesh=mesh,
        out_type=[
            jax.ShapeDtypeStruct((D, B), jnp.float32),
            jax.ShapeDtypeStruct((D, B), jnp.float32),
        ],
        scratch_types=[
            pltpu.VMEM((RPW,), jnp.int32),
            pltpu.VMEM((RPW,), jnp.int32),
            pltpu.VMEM((D * RPW,), jnp.int32),
            pltpu.VMEM((D * RPW,), jnp.float32),
            pltpu.SemaphoreType.DMA,
            pltpu.SemaphoreType.DMA,
        ],
        compiler_params=pltpu.CompilerParams(use_tc_tiling_on_sc=False),
    )
    def gather_kernel(uid_hbm, cid_hbm, utab_hbm, itab_hbm,
                      uout_hbm, cout_hbm,
                      uid_v, cid_v, idx_v, dst_v, semg, semo):
        wid = lax.axis_index("s") * NC + lax.axis_index("c")
        base = wid * RPW
        pltpu.sync_copy(uid_hbm.at[pl.ds(base, RPW)], uid_v)
        pltpu.sync_copy(cid_hbm.at[pl.ds(base, RPW)], cid_v)

        def one_table(ids_v, tab_hbm, out_hbm, nrows):
            def group(g, _):
                r16 = ids_v[pl.ds(g * 16, 16)]

                def dbody(d, _):
                    idx_v[pl.ds(d * RPW + g * 16, 16)] = r16 + d * nrows
                    return 0
                lax.fori_loop(0, D, dbody, 0, unroll=8)
                return 0
            lax.fori_loop(0, NG, group, 0)

            copies = []
            for q in range(D * RPW // CH):
                sl = pl.ds(q * CH, CH)
                copies.append(
                    pltpu.async_copy(tab_hbm.at[idx_v.at[sl]],
                                     dst_v.at[sl], semg))
            for cp in copies:
                cp.wait()

            outs = []
            for d in range(D):
                outs.append(
                    pltpu.async_copy(dst_v.at[pl.ds(d * RPW, RPW)],
                                     out_hbm.at[d, pl.ds(base, RPW)], semo))
            for cp in outs:
                cp.wait()

        one_table(uid_v, utab_hbm, uout_hbm, NU)
        one_table(cid_v, itab_hbm, cout_hbm, NI)

    return gather_kernel


def _mmT_body(w_ref, x_ref, b_ref, o_ref):
    encT = lax.dot_general(w_ref[...], x_ref[...], (((0,), (1,)), ((), ())),
                           preferred_element_type=jnp.float32)
    o_ref[...] = encT + b_ref[...]


def _tc_mmT(x, w, bcol):
    return pl.pallas_call(
        _mmT_body,
        grid=(B // BLK,),
        in_specs=[
            pl.BlockSpec((T, D), lambda i: (0, 0)),
            pl.BlockSpec((BLK, T), lambda i: (i, 0)),
            pl.BlockSpec((D, 1), lambda i: (0, 0)),
        ],
        out_specs=pl.BlockSpec((D, BLK), lambda i: (0, i)),
        out_shape=jax.ShapeDtypeStruct((D, B), jnp.float32),
    )(w, x, bcol)


def _dot_body(u_ref, c_ref, e_ref, o_ref):
    s = jnp.sum(u_ref[...] * (c_ref[...] + e_ref[...]), axis=0, keepdims=True)
    o_ref[...] = 1.0 / (1.0 + jnp.exp(-s))


def _tc_dot(uT, cT, encT):
    return pl.pallas_call(
        _dot_body,
        grid=(B // BLK,),
        in_specs=[
            pl.BlockSpec((D, BLK), lambda i: (0, i)),
            pl.BlockSpec((D, BLK), lambda i: (0, i)),
            pl.BlockSpec((D, BLK), lambda i: (0, i)),
        ],
        out_specs=pl.BlockSpec((1, BLK), lambda i: (0, i)),
        out_shape=jax.ShapeDtypeStruct((1, B), jnp.float32),
    )(uT, cT, encT)


def kernel(user_ids, content_ids, encoded_text, user_table, item_table,
           proj_W, proj_b):
    uid = user_ids.astype(jnp.int32)
    cid = content_ids.astype(jnp.int32)
    utab1 = user_table.T.reshape(-1)
    itab1 = item_table.T.reshape(-1)
    uT, cT = _sc_gather()(uid, cid, utab1, itab1)
    encT = _tc_mmT(encoded_text, proj_W, proj_b.reshape(D, 1))
    out = _tc_dot(uT, cT, encT)
    return out.reshape(B, 1)


# speedup vs baseline: 7.1133x; 7.1133x over previous
import functools

import jax
import jax.numpy as jnp
from jax import lax
from jax.experimental import pallas as pl
from jax.experimental.pallas import tpu as pltpu
from jax.experimental.pallas import tpu_sc as plsc

B = 16384
D = 64
T = 384
NC = 2
NS = 16
NW = NC * NS
RPW = B // NW
CH = 128
NCH = RPW // CH

BLK = 512


@functools.cache
def _sc_gather_one():
    mesh = plsc.VectorSubcoreMesh(core_axis_name="c", subcore_axis_name="s")

    @functools.partial(
        pl.kernel,
        mesh=mesh,
        out_type=jax.ShapeDtypeStruct((B, D), jnp.float32),
        scratch_types=[
            pltpu.VMEM((RPW,), jnp.int32),
            pltpu.VMEM((RPW, D), jnp.float32),
            pltpu.SemaphoreType.DMA,
        ],
        compiler_params=pltpu.CompilerParams(use_tc_tiling_on_sc=False),
    )
    def gather_kernel(ids_hbm, tab_hbm, out_hbm, ids_v, rows_v, sem):
        wid = lax.axis_index("s") * NC + lax.axis_index("c")
        base = wid * RPW
        pltpu.sync_copy(ids_hbm.at[pl.ds(base, RPW)], ids_v)
        copies = []
        for k in range(NCH):
            sl = pl.ds(k * CH, CH)
            copies.append(
                pltpu.async_copy(tab_hbm.at[ids_v.at[sl]], rows_v.at[sl], sem))
        for cp in copies:
            cp.wait()
        pltpu.sync_copy(rows_v, out_hbm.at[pl.ds(base, RPW)])

    return gather_kernel


def _mm_body(x_ref, w_ref, b_ref, o_ref):
    o_ref[...] = jnp.dot(x_ref[...], w_ref[...],
                         preferred_element_type=jnp.float32) + b_ref[...]


def _tc_matmul(x, w, b2):
    return pl.pallas_call(
        _mm_body,
        grid=(B // BLK,),
        in_specs=[
            pl.BlockSpec((BLK, T), lambda i: (i, 0)),
            pl.BlockSpec((T, D), lambda i: (0, 0)),
            pl.BlockSpec((1, D), lambda i: (0, 0)),
        ],
        out_specs=pl.BlockSpec((BLK, D), lambda i: (i, 0)),
        out_shape=jax.ShapeDtypeStruct((B, D), jnp.float32),
    )(x, w, b2)


def _dot_body(e_ref, u_ref, c_ref, o_ref):
    s = jnp.sum(u_ref[...] * (c_ref[...] + e_ref[...]), axis=1, keepdims=True)
    o_ref[...] = 1.0 / (1.0 + jnp.exp(-s))


def _tc_dot(enc, u_rows, c_rows):
    return pl.pallas_call(
        _dot_body,
        grid=(B // BLK,),
        in_specs=[
            pl.BlockSpec((BLK, D), lambda i: (i, 0)),
            pl.BlockSpec((BLK, D), lambda i: (i, 0)),
            pl.BlockSpec((BLK, D), lambda i: (i, 0)),
        ],
        out_specs=pl.BlockSpec((BLK, 1), lambda i: (i, 0)),
        out_shape=jax.ShapeDtypeStruct((B, 1), jnp.float32),
    )(enc, u_rows, c_rows)


def kernel(user_ids, content_ids, encoded_text, user_table, item_table,
           proj_W, proj_b):
    uid = user_ids.astype(jnp.int32)
    cid = content_ids.astype(jnp.int32)
    gather = _sc_gather_one()
    u_rows = gather(uid, user_table)
    c_rows = gather(cid, item_table)
    enc = _tc_matmul(encoded_text, proj_W, proj_b.reshape(1, D))
    return _tc_dot(enc, u_rows, c_rows)


# speedup vs baseline: 7.2247x; 1.0157x over previous
import functools

import jax
import jax.numpy as jnp
from jax import lax
from jax.experimental import pallas as pl
from jax.experimental.pallas import tpu as pltpu
from jax.experimental.pallas import tpu_sc as plsc

B = 16384
D = 64
T = 384
NC = 2
NS = 16
NW = NC * NS
RPW = B // NW
CH = 128
NCH = RPW // CH

BLK = 512


@functools.cache
def _sc_gather_one():
    mesh = plsc.VectorSubcoreMesh(core_axis_name="c", subcore_axis_name="s")

    @functools.partial(
        pl.kernel,
        mesh=mesh,
        out_type=jax.ShapeDtypeStruct((B, 2 * D), jnp.float32),
        scratch_types=[
            pltpu.VMEM((RPW,), jnp.int32),
            pltpu.VMEM((RPW, D), jnp.float32),
            pltpu.VMEM((RPW, 2 * D), jnp.float32),
            pltpu.SemaphoreType.DMA,
        ],
        compiler_params=pltpu.CompilerParams(use_tc_tiling_on_sc=False),
    )
    def gather_kernel(ids_hbm, tab_hbm, out_hbm, ids_v, rows_v, wide_v, sem):
        wid = lax.axis_index("s") * NC + lax.axis_index("c")
        base = wid * RPW
        pltpu.sync_copy(ids_hbm.at[pl.ds(base, RPW)], ids_v)
        copies = []
        for k in range(NCH):
            sl = pl.ds(k * CH, CH)
            copies.append(
                pltpu.async_copy(tab_hbm.at[ids_v.at[sl]], rows_v.at[sl], sem))
        for cp in copies:
            cp.wait()

        def widen(r, _):
            def col(k, _):
                wide_v[r, pl.ds(k * 16, 16)] = rows_v[r, pl.ds(k * 16, 16)]
                return 0
            lax.fori_loop(0, D // 16, col, 0, unroll=4)
            return 0
        lax.fori_loop(0, RPW, widen, 0)

        pltpu.sync_copy(wide_v, out_hbm.at[pl.ds(base, RPW)])

    return gather_kernel


def _mm_body(x_ref, w_ref, b_ref, o_ref):
    o_ref[...] = jnp.dot(x_ref[...], w_ref[...],
                         preferred_element_type=jnp.float32) + b_ref[...]


def _tc_matmul(x, w, b2):
    return pl.pallas_call(
        _mm_body,
        grid=(B // BLK,),
        in_specs=[
            pl.BlockSpec((BLK, T), lambda i: (i, 0)),
            pl.BlockSpec((T, D), lambda i: (0, 0)),
            pl.BlockSpec((1, D), lambda i: (0, 0)),
        ],
        out_specs=pl.BlockSpec((BLK, D), lambda i: (i, 0)),
        out_shape=jax.ShapeDtypeStruct((B, D), jnp.float32),
    )(x, w, b2)


def _dot_body(e_ref, u_ref, c_ref, o_ref):
    u = u_ref[:, :D]
    c = c_ref[:, :D]
    s = jnp.sum(u * (c + e_ref[...]), axis=1, keepdims=True)
    o_ref[...] = 1.0 / (1.0 + jnp.exp(-s))


def _tc_dot(enc, u_rows, c_rows):
    return pl.pallas_call(
        _dot_body,
        grid=(B // BLK,),
        in_specs=[
            pl.BlockSpec((BLK, D), lambda i: (i, 0)),
            pl.BlockSpec((BLK, 2 * D), lambda i: (i, 0)),
            pl.BlockSpec((BLK, 2 * D), lambda i: (i, 0)),
        ],
        out_specs=pl.BlockSpec((BLK, 1), lambda i: (i, 0)),
        out_shape=jax.ShapeDtypeStruct((B, 1), jnp.float32),
    )(enc, u_rows, c_rows)


def kernel(user_ids, content_ids, encoded_text, user_table, item_table,
           proj_W, proj_b):
    uid = user_ids.astype(jnp.int32)
    cid = content_ids.astype(jnp.int32)
    gather = _sc_gather_one()
    u_rows = gather(uid, user_table)
    c_rows = gather(cid, item_table)
    enc = _tc_matmul(encoded_text, proj_W, proj_b.reshape(1, D))
    return _tc_dot(enc, u_rows, c_rows)


# speedup vs baseline: 11.3238x; 1.5674x over previous
import functools

import jax
import jax.numpy as jnp
from jax import lax
from jax.experimental import pallas as pl
from jax.experimental.pallas import tpu as pltpu
from jax.experimental.pallas import tpu_sc as plsc

B = 16384
D = 64
T = 384
NC = 2
NS = 16
NW = NC * NS
RPW = B // NW
CH = 128
NCH = RPW // CH

BLK = 512


@functools.cache
def _sc_gather_one():
    mesh = plsc.VectorSubcoreMesh(core_axis_name="c", subcore_axis_name="s")

    @functools.partial(
        pl.kernel,
        mesh=mesh,
        out_type=jax.ShapeDtypeStruct((B, 2 * D), jnp.float32),
        scratch_types=[
            pltpu.VMEM((RPW,), jnp.int32),
            pltpu.VMEM((RPW // 2, D), jnp.float32),
            pltpu.VMEM((RPW // 2, 2 * D), jnp.float32),
            pltpu.SemaphoreType.DMA,
        ],
        compiler_params=pltpu.CompilerParams(use_tc_tiling_on_sc=True,
                                             needs_layout_passes=False),
    )
    def gather_kernel(ids_hbm, tab_hbm, out_hbm, ids_v, rows_v, wide_v, sem):
        wid = lax.axis_index("s") * NC + lax.axis_index("c")
        base = wid * RPW
        HPW = RPW // 2
        pltpu.sync_copy(ids_hbm.at[pl.ds(base, RPW)], ids_v)

        for h in range(2):
            def chunk(g, _):
                r16 = ids_v[pl.ds(h * HPW + g * 16, 16)]
                for j in range(16):
                    pltpu.async_copy(tab_hbm.at[pl.ds(r16[j], 1)],
                                     rows_v.at[pl.ds(g * 16 + j, 1)], sem)
                return 0
            lax.fori_loop(0, HPW // 16, chunk, 0)

            pltpu.make_async_copy(tab_hbm.at[pl.ds(0, HPW)], rows_v,
                                  sem).wait()

            def widen(r, _):
                def col(k, _):
                    wide_v[r, pl.ds(k * 16, 16)] = rows_v[r, pl.ds(k * 16, 16)]
                    return 0
                lax.fori_loop(0, D // 16, col, 0, unroll=4)
                return 0
            lax.fori_loop(0, HPW, widen, 0)

            pltpu.sync_copy(wide_v, out_hbm.at[pl.ds(base + h * HPW, HPW)])

    return gather_kernel


def _mm_body(x_ref, w_ref, b_ref, o_ref):
    o_ref[...] = jnp.dot(x_ref[...], w_ref[...],
                         preferred_element_type=jnp.float32) + b_ref[...]


def _tc_matmul(x, w, b2):
    return pl.pallas_call(
        _mm_body,
        grid=(B // BLK,),
        in_specs=[
            pl.BlockSpec((BLK, T), lambda i: (i, 0)),
            pl.BlockSpec((T, D), lambda i: (0, 0)),
            pl.BlockSpec((1, D), lambda i: (0, 0)),
        ],
        out_specs=pl.BlockSpec((BLK, D), lambda i: (i, 0)),
        out_shape=jax.ShapeDtypeStruct((B, D), jnp.float32),
    )(x, w, b2)


def _dot_body(e_ref, u_ref, c_ref, o_ref):
    u = u_ref[:, :D]
    c = c_ref[:, :D]
    s = jnp.sum(u * (c + e_ref[...]), axis=1, keepdims=True)
    o_ref[...] = 1.0 / (1.0 + jnp.exp(-s))


def _tc_dot(enc, u_rows, c_rows):
    return pl.pallas_call(
        _dot_body,
        grid=(B // BLK,),
        in_specs=[
            pl.BlockSpec((BLK, D), lambda i: (i, 0)),
            pl.BlockSpec((BLK, 2 * D), lambda i: (i, 0)),
            pl.BlockSpec((BLK, 2 * D), lambda i: (i, 0)),
        ],
        out_specs=pl.BlockSpec((BLK, 1), lambda i: (i, 0)),
        out_shape=jax.ShapeDtypeStruct((B, 1), jnp.float32),
    )(enc, u_rows, c_rows)


def kernel(user_ids, content_ids, encoded_text, user_table, item_table,
           proj_W, proj_b):
    uid = user_ids.astype(jnp.int32)
    cid = content_ids.astype(jnp.int32)
    gather = _sc_gather_one()
    u_rows = gather(uid, user_table)
    c_rows = gather(cid, item_table)
    enc = _tc_matmul(encoded_text, proj_W, proj_b.reshape(1, D))
    return _tc_dot(enc, u_rows, c_rows)
